# baseline (speedup 1.0000x reference)
import jax
import jax.numpy as jnp
from jax import lax
from jax.experimental import pallas as pl
from jax.experimental.pallas import tpu as pltpu
from jax.experimental.pallas import tpu_sc as plsc

NC, NS, L = 2, 16, 16
NW = NC * NS
BATCH, SEQ, HID = 16384, 200, 64
SEQ_PER_W = BATCH // NW
NSEQ_CHUNK = 1
N_ITERS = SEQ_PER_W // NSEQ_CHUNK
IDX_ROW = 100
G = NSEQ_CHUNK * SEQ // IDX_ROW
NBUF = 8
K = 3


def _body(idx_hbm, wtok_hbm, wpos_hbm, out_hbm, idx_v, rows_v, pos_v,
          gsem, osem):
    c = lax.axis_index("c")
    s = lax.axis_index("s")
    wid = s * NC + c
    pltpu.sync_copy(wpos_hbm.at[pl.ds(0, SEQ)], pos_v)
    seq0 = wid * SEQ_PER_W

    def gather_pairs(b):
        for g in range(G):
            yield (idx_v.at[b, g], rows_v.at[b, pl.ds(g * IDX_ROW, IDX_ROW)])

    def fire_gathers(it, b):
        row0 = (seq0 + it * NSEQ_CHUNK) * (SEQ // IDX_ROW)
        pltpu.sync_copy(idx_hbm.at[pl.ds(row0, G)], idx_v.at[b])
        for isrc, rdst in gather_pairs(b):
            pltpu.async_copy(wtok_hbm.at[isrc], rdst, gsem.at[b])

    def wait_gathers(b):
        for isrc, rdst in gather_pairs(b):
            pltpu.make_async_copy(wtok_hbm.at[isrc], rdst, gsem.at[b]).wait()

    def write_pairs(b, it):
        for n in range(NSEQ_CHUNK):
            yield (rows_v.at[b, pl.ds(n * SEQ, SEQ)],
                   out_hbm.at[seq0 + it * NSEQ_CHUNK + n, pl.ds(0, SEQ),
                              pl.ds(0, HID)])

    def wait_write(b, it):
        for src, dst in write_pairs(b, it):
            pltpu.make_async_copy(src, dst, osem.at[b]).wait()

    def add_pos(b):
        @plsc.parallel_loop(0, SEQ, unroll=2)
        def _add_row(j):
            for q in range(HID // L):
                sl = pl.ds(q * L, L)
                p = pos_v[j, sl]
                for n in range(NSEQ_CHUNK):
                    plsc.addupdate(rows_v.at[b, n * SEQ + j, sl], p)

    for b in range(K):
        fire_gathers(b, b)

    def superstep(ss, carry):
        for b in range(NBUF):
            it = ss * NBUF + b
            nxt = it + K
            pb = (b + K) % NBUF
            @pl.when(jnp.logical_and(nxt >= NBUF, nxt < N_ITERS))
            def _():
                wait_write(pb, nxt - NBUF)
            @pl.when(nxt < N_ITERS)
            def _():
                fire_gathers(nxt, pb)
            wait_gathers(b)
            add_pos(b)
            for src_, dst_ in write_pairs(b, it):
                pltpu.async_copy(src_, dst_, osem.at[b])
        return carry

    lax.fori_loop(0, N_ITERS // NBUF, superstep, 0)

    for it in range(N_ITERS - K, N_ITERS):
        wait_write(it % NBUF, it)


_mesh = plsc.VectorSubcoreMesh(core_axis_name="c", subcore_axis_name="s")

_gather_add = pl.kernel(
    _body,
    mesh=_mesh,
    compiler_params=pltpu.CompilerParams(use_tc_tiling_on_sc=False),
    out_type=jax.ShapeDtypeStruct((BATCH, SEQ, 128), jnp.float32),
    scratch_types=[
        pltpu.VMEM((NBUF, G, IDX_ROW), jnp.int32),
        pltpu.VMEM((NBUF, NSEQ_CHUNK * SEQ, HID), jnp.float32),
        pltpu.VMEM((SEQ, HID), jnp.float32),
        pltpu.SemaphoreType.DMA((NBUF,)),
        pltpu.SemaphoreType.DMA((NBUF,)),
    ],
)


def kernel(input_X, W_tok, W_pos):
    idx = input_X.astype(jnp.int32).reshape(BATCH * SEQ // IDX_ROW, IDX_ROW)
    o = _gather_add(idx, W_tok, W_pos)
    return o[:, :, 0:HID]


# speedup vs baseline: 1.1653x; 1.1653x over previous
import jax
import jax.numpy as jnp
from jax import lax
from jax.experimental import pallas as pl
from jax.experimental.pallas import tpu as pltpu
from jax.experimental.pallas import tpu_sc as plsc

NC, NS, L = 2, 16, 16
NW = NC * NS
BATCH, SEQ, HID = 16384, 200, 64
SEQ_PER_W = BATCH // NW
NSEQ_CHUNK = 2
N_ITERS = SEQ_PER_W // NSEQ_CHUNK
IDX_ROW = 104
IDX_OFF = (0, 96)
NBUF = 4
K = 2


def _body(idx_hbm, wtok_hbm, wpos_hbm, out_hbm, idx_v, rows_v, pos_v,
          gsem, osem):
    c = lax.axis_index("c")
    s = lax.axis_index("s")
    wid = s * NC + c
    pltpu.sync_copy(wpos_hbm.at[pl.ds(0, SEQ)], pos_v)
    seq0 = wid * SEQ_PER_W

    def gather_pairs(b):
        for n in range(NSEQ_CHUNK):
            for off in IDX_OFF:
                yield (idx_v.at[b, n, pl.ds(off, IDX_ROW)],
                       rows_v.at[b, pl.ds(n * SEQ + off, IDX_ROW)])

    def fire_gathers(it, b):
        seq = seq0 + it * NSEQ_CHUNK
        pltpu.sync_copy(idx_hbm.at[pl.ds(seq, NSEQ_CHUNK)], idx_v.at[b])
        for isrc, rdst in gather_pairs(b):
            pltpu.async_copy(wtok_hbm.at[isrc], rdst, gsem.at[b])

    def wait_gathers(b):
        for isrc, rdst in gather_pairs(b):
            pltpu.make_async_copy(wtok_hbm.at[isrc], rdst, gsem.at[b]).wait()

    def write_pairs(b, it):
        for n in range(NSEQ_CHUNK):
            yield (rows_v.at[b, pl.ds(n * SEQ, SEQ)],
                   out_hbm.at[seq0 + it * NSEQ_CHUNK + n, pl.ds(0, SEQ),
                              pl.ds(0, HID)])

    def wait_write(b, it):
        for src, dst in write_pairs(b, it):
            pltpu.make_async_copy(src, dst, osem.at[b]).wait()

    def add_pos(b):
        @plsc.parallel_loop(0, SEQ, unroll=2)
        def _add_row(j):
            for q in range(HID // L):
                sl = pl.ds(q * L, L)
                p = pos_v[j, sl]
                for n in range(NSEQ_CHUNK):
                    plsc.addupdate(rows_v.at[b, n * SEQ + j, sl], p)

    for b in range(K):
        fire_gathers(b, b)

    def superstep(ss, carry):
        for b in range(NBUF):
            it = ss * NBUF + b
            nxt = it + K
            pb = (b + K) % NBUF
            @pl.when(jnp.logical_and(nxt >= NBUF, nxt < N_ITERS))
            def _():
                wait_write(pb, nxt - NBUF)
            @pl.when(nxt < N_ITERS)
            def _():
                fire_gathers(nxt, pb)
            wait_gathers(b)
            add_pos(b)
            for src_, dst_ in write_pairs(b, it):
                pltpu.async_copy(src_, dst_, osem.at[b])
        return carry

    lax.fori_loop(0, N_ITERS // NBUF, superstep, 0)

    for it in range(N_ITERS - K, N_ITERS):
        wait_write(it % NBUF, it)


_mesh = plsc.VectorSubcoreMesh(core_axis_name="c", subcore_axis_name="s")

_gather_add = pl.kernel(
    _body,
    mesh=_mesh,
    compiler_params=pltpu.CompilerParams(use_tc_tiling_on_sc=False),
    out_type=jax.ShapeDtypeStruct((BATCH, SEQ, 128), jnp.float32),
    scratch_types=[
        pltpu.VMEM((NBUF, NSEQ_CHUNK, SEQ), jnp.int32),
        pltpu.VMEM((NBUF, NSEQ_CHUNK * SEQ, HID), jnp.float32),
        pltpu.VMEM((SEQ, HID), jnp.float32),
        pltpu.SemaphoreType.DMA((NBUF,)),
        pltpu.SemaphoreType.DMA((NBUF,)),
    ],
)


def kernel(input_X, W_tok, W_pos):
    o = _gather_add(input_X.astype(jnp.int32), W_tok, W_pos)
    return o[:, :, 0:HID]
